# Initial kernel scaffold; baseline (speedup 1.0000x reference)
import functools
import math

import jax
import jax.numpy as jnp
from jax.experimental import pallas as pl

_N = 10000
_D = 128
_G = 64
_LOG2 = math.log(2.0)

_B = 512
_NP = ((_N + _B - 1) // _B) * _B


def _loss_body(zl_ref, gg_ref, bm_ref, s1_ref, s2_ref, p1_ref, p2_ref):
    i = pl.program_id(0)

    @pl.when(i == 0)
    def _init():
        s1_ref[0, 0] = 0.0
        s2_ref[0, 0] = 0.0
        p1_ref[0, 0] = 0.0
        p2_ref[0, 0] = 0.0

    zl = zl_ref[...]
    gg = gg_ref[...]
    b = bm_ref[...]
    r = jnp.dot(zl, gg.T, preferred_element_type=jnp.float32)
    colid = jax.lax.broadcasted_iota(jnp.float32, (_B, _G), 1)
    pos = b == colid
    valid = b >= 0.0
    sp = jnp.maximum(-r, 0.0) + jnp.log1p(jnp.exp(-jnp.abs(r)))
    zero = jnp.zeros_like(r)
    s1_ref[0, 0] += jnp.sum(jnp.where(valid, sp, zero))
    s2_ref[0, 0] += jnp.sum(jnp.where(valid, r, zero))
    p1_ref[0, 0] += jnp.sum(jnp.where(pos, sp, zero))
    p2_ref[0, 0] += jnp.sum(jnp.where(pos, r, zero))


def _loss_sums(zl, gg, bmask):
    grid = _NP // _B
    out = pl.pallas_call(
        _loss_body,
        grid=(grid,),
        in_specs=[
            pl.BlockSpec((_B, _D), lambda i: (i, 0)),
            pl.BlockSpec((_G, _D), lambda i: (0, 0)),
            pl.BlockSpec((_B, _G), lambda i: (i, 0)),
        ],
        out_specs=[pl.BlockSpec((1, 1), lambda i: (0, 0))] * 4,
        out_shape=[jax.ShapeDtypeStruct((1, 1), jnp.float32)] * 4,
    )(zl, gg, bmask)
    return [o[0, 0] for o in out]


def _prelu(x, a):
    return jnp.where(x >= 0, x, a * x)


def _mlp(p, x):
    h = _prelu(x @ p["fc1"]["W"] + p["fc1"]["b"], p["a1"])
    h = _prelu(h @ p["fc2"]["W"] + p["fc2"]["b"], p["a2"])
    h = _prelu(h @ p["fc3"]["W"] + p["fc3"]["b"], p["a3"])
    return h + (x @ p["sc"]["W"] + p["sc"]["b"])


def _branch_loss(enc_p, local_p, global_p, x, edge_index, batch):
    src = edge_index[0]
    dst = edge_index[1]
    deg = jax.ops.segment_sum(jnp.ones((edge_index.shape[1],), jnp.float32),
                              dst, num_segments=_N)
    deg = jnp.clip(deg, 1.0, None)
    rdeg = jax.lax.rsqrt(deg)
    xs = x * rdeg[:, None]
    s = jax.ops.segment_sum(xs[src], dst, num_segments=_N)
    agg = s * rdeg[:, None]
    z = jax.nn.relu(agg @ enc_p["W"] + enc_p["b"])
    g = jax.ops.segment_sum(z, batch, num_segments=_G)
    zl = _mlp(local_p, z)
    gg = _mlp(global_p, g)

    zl_pad = jnp.pad(zl, ((0, _NP - _N), (0, 0)))
    bm = jnp.pad(batch.astype(jnp.float32), (0, _NP - _N),
                 constant_values=-1.0)
    bm = jnp.broadcast_to(bm[:, None], (_NP, _G))
    s1, s2, p1, p2 = _loss_sums(zl_pad, gg, bm)

    e_pos = (_N * _LOG2 - p1) / _N
    e_neg = ((s1 + s2 - _N * _G * _LOG2) - (p1 + p2 - _N * _LOG2)) / (_N * (_G - 1))
    return 2.0 * (e_neg - e_pos)


def kernel(x1, x2, x3, edge_index1, edge_index2, edge_index3,
           batch1, batch2, batch3, params):
    l1 = _branch_loss(params["enc1"], params["local"], params["global"],
                      x1, edge_index1, batch1)
    l2 = _branch_loss(params["enc2"], params["local"], params["global"],
                      x2, edge_index2, batch2)
    l3 = _branch_loss(params["enc3"], params["local"], params["global"],
                      x3, edge_index3, batch3)
    return l1 + l2 + l3


# speedup vs baseline: 3.6677x; 3.6677x over previous
import functools
import math

import jax
import jax.numpy as jnp
from jax.experimental import pallas as pl

_N = 10000
_D = 128
_G = 64
_LOG2 = math.log(2.0)

_B = 512
_NP = ((_N + _B - 1) // _B) * _B


def _loss_body(zl_ref, gg_ref, bm_ref, s1_ref, s2_ref, p1_ref, p2_ref):
    i = pl.program_id(0)

    @pl.when(i == 0)
    def _init():
        s1_ref[...] = jnp.zeros((1, 1), jnp.float32)
        s2_ref[...] = jnp.zeros((1, 1), jnp.float32)
        p1_ref[...] = jnp.zeros((1, 1), jnp.float32)
        p2_ref[...] = jnp.zeros((1, 1), jnp.float32)

    zl = zl_ref[...]
    gg = gg_ref[...]
    b = bm_ref[...]
    r = jnp.dot(zl, gg.T, preferred_element_type=jnp.float32)
    colid = jax.lax.broadcasted_iota(jnp.int32, (_B, _G), 1)
    pos = b == colid
    valid = b >= 0
    sp = jnp.maximum(-r, 0.0) + jnp.log1p(jnp.exp(-jnp.abs(r)))
    zero = jnp.zeros_like(r)
    s1_ref[...] += jnp.sum(jnp.where(valid, sp, zero), keepdims=True)
    s2_ref[...] += jnp.sum(jnp.where(valid, r, zero), keepdims=True)
    p1_ref[...] += jnp.sum(jnp.where(pos, sp, zero), keepdims=True)
    p2_ref[...] += jnp.sum(jnp.where(pos, r, zero), keepdims=True)


def _loss_sums(zl, gg, bmask):
    grid = _NP // _B
    out = pl.pallas_call(
        _loss_body,
        grid=(grid,),
        in_specs=[
            pl.BlockSpec((_B, _D), lambda i: (i, 0)),
            pl.BlockSpec((_G, _D), lambda i: (0, 0)),
            pl.BlockSpec((_B, _G), lambda i: (i, 0)),
        ],
        out_specs=[pl.BlockSpec((1, 1), lambda i: (0, 0))] * 4,
        out_shape=[jax.ShapeDtypeStruct((1, 1), jnp.float32)] * 4,
    )(zl, gg, bmask)
    return [o[0, 0] for o in out]


def _prelu(x, a):
    return jnp.where(x >= 0, x, a * x)


def _mlp(p, x):
    h = _prelu(x @ p["fc1"]["W"] + p["fc1"]["b"], p["a1"])
    h = _prelu(h @ p["fc2"]["W"] + p["fc2"]["b"], p["a2"])
    h = _prelu(h @ p["fc3"]["W"] + p["fc3"]["b"], p["a3"])
    return h + (x @ p["sc"]["W"] + p["sc"]["b"])


def _branch_loss(enc_p, local_p, global_p, x, edge_index, batch):
    src = edge_index[0]
    dst = edge_index[1]
    deg = jax.ops.segment_sum(jnp.ones((edge_index.shape[1],), jnp.float32),
                              dst, num_segments=_N)
    deg = jnp.clip(deg, 1.0, None)
    rdeg = jax.lax.rsqrt(deg)
    xs = x * rdeg[:, None]
    s = jax.ops.segment_sum(xs[src], dst, num_segments=_N)
    agg = s * rdeg[:, None]
    z = jax.nn.relu(agg @ enc_p["W"] + enc_p["b"])
    g = jax.ops.segment_sum(z, batch, num_segments=_G)
    zl = _mlp(local_p, z)
    gg = _mlp(global_p, g)

    zl_pad = jnp.pad(zl, ((0, _NP - _N), (0, 0)))
    bm = jnp.pad(batch, (0, _NP - _N), constant_values=-1)
    bm = jnp.broadcast_to(bm[:, None], (_NP, _G))
    s1, s2, p1, p2 = _loss_sums(zl_pad, gg, bm)

    e_pos = (_N * _LOG2 - p1) / _N
    e_neg = ((s1 + s2 - _N * _G * _LOG2) - (p1 + p2 - _N * _LOG2)) / (_N * (_G - 1))
    return 2.0 * (e_neg - e_pos)


def kernel(x1, x2, x3, edge_index1, edge_index2, edge_index3,
           batch1, batch2, batch3, params):
    l1 = _branch_loss(params["enc1"], params["local"], params["global"],
                      x1, edge_index1, batch1)
    l2 = _branch_loss(params["enc2"], params["local"], params["global"],
                      x2, edge_index2, batch2)
    l3 = _branch_loss(params["enc3"], params["local"], params["global"],
                      x3, edge_index3, batch3)
    return l1 + l2 + l3


# speedup vs baseline: 20.6130x; 5.6202x over previous
import functools
import math

import jax
import jax.numpy as jnp
from jax import lax
from jax.experimental import pallas as pl
from jax.experimental.pallas import tpu as pltpu
import jax.experimental.pallas.tpu_sc as plsc

_N = 10000
_E = 320000
_D = 128
_G = 64
_LOG2 = math.log(2.0)

_NC, _NS = 2, 16
_NW = _NC * _NS
_EW = _E // _NW
_CH = 125
_NCH = _EW // _CH
_RT = 640

_B = 1000
_NB = _N // _B

_sc_mesh = plsc.VectorSubcoreMesh(
    core_axis_name="c", subcore_axis_name="s", num_cores=_NC, num_subcores=_NS)


def _row_slab_copy(s, src, dst):
    @pl.when(s < _NS - 1)
    def _main():
        pltpu.sync_copy(src.at[pl.ds(s * _RT, _RT)],
                        dst.at[pl.ds(s * _RT, _RT)])

    @pl.when(s == _NS - 1)
    def _tail():
        pltpu.sync_copy(src.at[pl.ds(_RT * (_NS - 1), _N - _RT * (_NS - 1))],
                        dst.at[pl.ds(_RT * (_NS - 1), _N - _RT * (_NS - 1))])


def _deg_body(dst2, ones_h, zeros128, out, idx_v, ones_v, deg_sh):
    c = lax.axis_index("c")
    s = lax.axis_index("s")
    wid = s * _NC + c
    _row_slab_copy(s, zeros128, deg_sh)
    pltpu.sync_copy(ones_h, ones_v)
    pltpu.sync_copy(dst2.at[pl.ds(wid * _NCH, _NCH)], idx_v)
    plsc.subcore_barrier()

    def body(k, carry):
        pltpu.sync_copy(ones_v, deg_sh.at[idx_v.at[k]], add=True)
        return carry

    lax.fori_loop(0, _NCH, body, 0)
    plsc.subcore_barrier()
    _row_slab_copy(s, deg_sh, out.at[c])


_deg_call = pl.kernel(
    _deg_body,
    out_type=jax.ShapeDtypeStruct((_NC, _N, _D), jnp.float32),
    mesh=_sc_mesh,
    scratch_types=[
        pltpu.VMEM((_NCH, _CH), jnp.int32),
        pltpu.VMEM((_CH, _D), jnp.float32),
        pltpu.VMEM_SHARED((_N, _D), jnp.float32),
    ],
)


def _agg_body(src2, dst2, xs, zeros128, out, isrc_v, idst_v, rows_v, sem,
              agg_sh):
    c = lax.axis_index("c")
    s = lax.axis_index("s")
    wid = s * _NC + c
    _row_slab_copy(s, zeros128, agg_sh)
    pltpu.sync_copy(src2.at[pl.ds(wid * _NCH, _NCH)], isrc_v)
    pltpu.sync_copy(dst2.at[pl.ds(wid * _NCH, _NCH)], idst_v)
    plsc.subcore_barrier()

    def body(k, carry):
        pltpu.async_copy(xs.at[isrc_v.at[k]], rows_v, sem).wait()
        pltpu.sync_copy(rows_v, agg_sh.at[idst_v.at[k]], add=True)
        return carry

    lax.fori_loop(0, _NCH, body, 0)
    plsc.subcore_barrier()
    _row_slab_copy(s, agg_sh, out.at[c])


_agg_call = pl.kernel(
    _agg_body,
    out_type=jax.ShapeDtypeStruct((_NC, _N, _D), jnp.float32),
    mesh=_sc_mesh,
    scratch_types=[
        pltpu.VMEM((_NCH, _CH), jnp.int32),
        pltpu.VMEM((_NCH, _CH), jnp.int32),
        pltpu.VMEM((_CH, _D), jnp.float32),
        pltpu.SemaphoreType.DMA,
        pltpu.VMEM_SHARED((_N, _D), jnp.float32),
    ],
)


def _xs_body(x_ref, d0_ref, d1_ref, xs_ref):
    deg = jnp.maximum(d0_ref[...][:, :1] + d1_ref[...][:, :1], 1.0)
    xs_ref[...] = x_ref[...] * lax.rsqrt(deg)


def _xs_scale(x, d0, d1):
    return pl.pallas_call(
        _xs_body,
        grid=(_NB,),
        in_specs=[
            pl.BlockSpec((_B, _D), lambda i: (i, 0)),
            pl.BlockSpec((_B, _D), lambda i: (i, 0)),
            pl.BlockSpec((_B, _D), lambda i: (i, 0)),
        ],
        out_specs=pl.BlockSpec((_B, _D), lambda i: (i, 0)),
        out_shape=jax.ShapeDtypeStruct((_N, _D), jnp.float32),
    )(x, d0, d1)


def _mlp_block(x, W_ref, b_ref, a_ref):
    h = x
    for j in range(3):
        h = jnp.dot(h, W_ref[j], preferred_element_type=jnp.float32) \
            + b_ref[j:j + 1, :]
        a = a_ref[j:j + 1, :]
        h = jnp.where(h >= 0, h, a * h)
    return h + jnp.dot(x, W_ref[3], preferred_element_type=jnp.float32) \
        + b_ref[3:4, :]


def _enc_body(s0, s1, d0, d1, bm, encW, encb, LW, Lb, La, zl_ref, g_ref):
    i = pl.program_id(0)

    @pl.when(i == 0)
    def _init():
        g_ref[...] = jnp.zeros((_G, _D), jnp.float32)

    deg = jnp.maximum(d0[...][:, :1] + d1[...][:, :1], 1.0)
    rdeg = lax.rsqrt(deg)
    agg = (s0[...] + s1[...]) * rdeg
    z = jnp.maximum(
        jnp.dot(agg, encW[...], preferred_element_type=jnp.float32)
        + encb[...], 0.0)
    colid = lax.broadcasted_iota(jnp.int32, (_B, _G), 1)
    oh = (bm[...] == colid).astype(jnp.float32)
    g_ref[...] += lax.dot_general(oh, z, (((0,), (0,)), ((), ())),
                                  preferred_element_type=jnp.float32)
    zl_ref[...] = _mlp_block(z, LW, Lb, La)


def _enc_call(s0, s1, d0, d1, bm, encW, encb, LW, Lb, La):
    return pl.pallas_call(
        _enc_body,
        grid=(_NB,),
        in_specs=[
            pl.BlockSpec((_B, _D), lambda i: (i, 0)),
            pl.BlockSpec((_B, _D), lambda i: (i, 0)),
            pl.BlockSpec((_B, _D), lambda i: (i, 0)),
            pl.BlockSpec((_B, _D), lambda i: (i, 0)),
            pl.BlockSpec((_B, _G), lambda i: (i, 0)),
            pl.BlockSpec((_D, _D), lambda i: (0, 0)),
            pl.BlockSpec((1, _D), lambda i: (0, 0)),
            pl.BlockSpec((4, _D, _D), lambda i: (0, 0, 0)),
            pl.BlockSpec((4, _D), lambda i: (0, 0)),
            pl.BlockSpec((3, _D), lambda i: (0, 0)),
        ],
        out_specs=[
            pl.BlockSpec((_B, _D), lambda i: (i, 0)),
            pl.BlockSpec((_G, _D), lambda i: (0, 0)),
        ],
        out_shape=[
            jax.ShapeDtypeStruct((_N, _D), jnp.float32),
            jax.ShapeDtypeStruct((_G, _D), jnp.float32),
        ],
    )(s0, s1, d0, d1, bm, encW, encb, LW, Lb, La)


def _loss_body(zl, g, GW, Gb, Ga, bm, s1_ref, s2_ref, p1_ref, p2_ref, gg_s):
    i = pl.program_id(0)

    @pl.when(i == 0)
    def _init():
        gg_s[...] = _mlp_block(g[...], GW, Gb, Ga)
        s1_ref[...] = jnp.zeros((1, 1), jnp.float32)
        s2_ref[...] = jnp.zeros((1, 1), jnp.float32)
        p1_ref[...] = jnp.zeros((1, 1), jnp.float32)
        p2_ref[...] = jnp.zeros((1, 1), jnp.float32)

    r = lax.dot_general(zl[...], gg_s[...], (((1,), (1,)), ((), ())),
                        preferred_element_type=jnp.float32)
    colid = lax.broadcasted_iota(jnp.int32, (_B, _G), 1)
    pos = bm[...] == colid
    sp = jnp.maximum(-r, 0.0) + jnp.log1p(jnp.exp(-jnp.abs(r)))
    zero = jnp.zeros_like(r)
    s1_ref[...] += jnp.sum(sp, keepdims=True)
    s2_ref[...] += jnp.sum(r, keepdims=True)
    p1_ref[...] += jnp.sum(jnp.where(pos, sp, zero), keepdims=True)
    p2_ref[...] += jnp.sum(jnp.where(pos, r, zero), keepdims=True)


def _loss_call(zl, g, GW, Gb, Ga, bm):
    return pl.pallas_call(
        _loss_body,
        grid=(_NB,),
        in_specs=[
            pl.BlockSpec((_B, _D), lambda i: (i, 0)),
            pl.BlockSpec((_G, _D), lambda i: (0, 0)),
            pl.BlockSpec((4, _D, _D), lambda i: (0, 0, 0)),
            pl.BlockSpec((4, _D), lambda i: (0, 0)),
            pl.BlockSpec((3, _D), lambda i: (0, 0)),
            pl.BlockSpec((_B, _G), lambda i: (i, 0)),
        ],
        out_specs=[pl.BlockSpec((1, 1), lambda i: (0, 0))] * 4,
        out_shape=[jax.ShapeDtypeStruct((1, 1), jnp.float32)] * 4,
        scratch_shapes=[pltpu.VMEM((_G, _D), jnp.float32)],
    )(zl, g, GW, Gb, Ga, bm)


def _stack_mlp(p):
    W = jnp.stack([p["fc1"]["W"], p["fc2"]["W"], p["fc3"]["W"], p["sc"]["W"]])
    b = jnp.stack([p["fc1"]["b"], p["fc2"]["b"], p["fc3"]["b"], p["sc"]["b"]])
    a = jnp.stack([jnp.broadcast_to(p["a1"], (_D,)),
                   jnp.broadcast_to(p["a2"], (_D,)),
                   jnp.broadcast_to(p["a3"], (_D,))])
    return W, b, a


def _branch_loss(enc_p, Lwba, Gwba, x, edge_index, batch, ones128,
                 zeros128):
    src2 = edge_index[0].reshape(_E // _CH, _CH)
    dst2 = edge_index[1].reshape(_E // _CH, _CH)
    deg2 = _deg_call(dst2, ones128, zeros128)
    d0, d1 = deg2[0], deg2[1]
    xs = _xs_scale(x, d0, d1)
    s2o = _agg_call(src2, dst2, xs, zeros128)
    bm = jnp.broadcast_to(batch[:, None], (_N, _G))
    encb = enc_p["b"].reshape(1, _D)
    zl, g = _enc_call(s2o[0], s2o[1], d0, d1, bm, enc_p["W"], encb, *Lwba)
    s1, s2, p1, p2 = [o[0, 0] for o in _loss_call(zl, g, *Gwba, bm)]

    e_pos = (_N * _LOG2 - p1) / _N
    e_neg = ((s1 + s2 - _N * _G * _LOG2) - (p1 + p2 - _N * _LOG2)) \
        / (_N * (_G - 1))
    return 2.0 * (e_neg - e_pos)


def kernel(x1, x2, x3, edge_index1, edge_index2, edge_index3,
           batch1, batch2, batch3, params):
    ones128 = jnp.ones((_CH, _D), jnp.float32)
    zeros128 = jnp.zeros((_N, _D), jnp.float32)
    Lwba = _stack_mlp(params["local"])
    Gwba = _stack_mlp(params["global"])
    l1 = _branch_loss(params["enc1"], Lwba, Gwba, x1, edge_index1, batch1,
                      ones128, zeros128)
    l2 = _branch_loss(params["enc2"], Lwba, Gwba, x2, edge_index2, batch2,
                      ones128, zeros128)
    l3 = _branch_loss(params["enc3"], Lwba, Gwba, x3, edge_index3, batch3,
                      ones128, zeros128)
    return l1 + l2 + l3


# speedup vs baseline: 26.6554x; 1.2931x over previous
import functools
import math

import jax
import jax.numpy as jnp
from jax import lax
from jax.experimental import pallas as pl
from jax.experimental.pallas import tpu as pltpu
import jax.experimental.pallas.tpu_sc as plsc

_N = 10000
_E = 320000
_D = 128
_G = 64
_LOG2 = math.log(2.0)

_NC, _NS = 2, 16
_NW = _NC * _NS
_EW = _E // _NW
_CH = 125
_NCH = _EW // _CH
_HNCH = _NCH // 2
_RT = 640

_B = 1000
_NB = _N // _B

_sc_mesh = plsc.VectorSubcoreMesh(
    core_axis_name="c", subcore_axis_name="s", num_cores=_NC, num_subcores=_NS)


def _row_slab_copy(s, src, dst):
    @pl.when(s < _NS - 1)
    def _main():
        pltpu.sync_copy(src.at[pl.ds(s * _RT, _RT)],
                        dst.at[pl.ds(s * _RT, _RT)])

    @pl.when(s == _NS - 1)
    def _tail():
        pltpu.sync_copy(src.at[pl.ds(_RT * (_NS - 1), _N - _RT * (_NS - 1))],
                        dst.at[pl.ds(_RT * (_NS - 1), _N - _RT * (_NS - 1))])


_TAIL = _N - _RT * (_NS - 1)


def _deg_body(dst2, ones_h, zeros128, out, idx_v, ones_v, sem, deg_sh):
    c = lax.axis_index("c")
    s = lax.axis_index("s")
    wid = s * _NC + c
    _row_slab_copy(s, zeros128, deg_sh)
    pltpu.sync_copy(ones_h, ones_v)
    pltpu.sync_copy(dst2.at[pl.ds(wid * _NCH, _NCH)], idx_v)
    plsc.subcore_barrier()

    def body(k8, carry):
        descs = [
            pltpu.async_copy(ones_v, deg_sh.at[idx_v.at[k8 * 8 + b]], sem,
                             add=True)
            for b in range(8)
        ]
        for d in descs:
            d.wait()
        return carry

    lax.fori_loop(0, _NCH // 8, body, 0)
    plsc.subcore_barrier()
    _row_slab_copy(s, deg_sh, out.at[c])


_deg_call = pl.kernel(
    _deg_body,
    out_type=jax.ShapeDtypeStruct((_NC, _N, _D), jnp.float32),
    mesh=_sc_mesh,
    scratch_types=[
        pltpu.VMEM((_NCH, _CH), jnp.int32),
        pltpu.VMEM((_CH, _D), jnp.float32),
        pltpu.SemaphoreType.DMA,
        pltpu.VMEM_SHARED((_N, _D), jnp.float32),
    ],
)


def _agg_body(src2, dst2, xs, zeros128, out, isrc_v, idst_v, rows_a, rows_b,
              sem_a, sem_b, agg_sh):
    c = lax.axis_index("c")
    s = lax.axis_index("s")
    wid = s * _NC + c
    _row_slab_copy(s, zeros128, agg_sh)
    plsc.subcore_barrier()

    def half(h, carry):
        base = wid * _NCH + h * _HNCH
        pltpu.sync_copy(src2.at[pl.ds(base, _HNCH)], isrc_v)
        pltpu.sync_copy(dst2.at[pl.ds(base, _HNCH)], idst_v)
        pltpu.async_copy(xs.at[isrc_v.at[0]], rows_a, sem_a)

        def body(k, carry2):
            k2 = 2 * k
            pltpu.async_copy(xs.at[isrc_v.at[k2 + 1]], rows_b, sem_b)
            pltpu.make_async_copy(xs.at[isrc_v.at[k2]], rows_a, sem_a).wait()
            pltpu.sync_copy(rows_a, agg_sh.at[idst_v.at[k2]], add=True)

            @pl.when(k2 + 2 < _HNCH)
            def _next():
                pltpu.async_copy(xs.at[isrc_v.at[k2 + 2]], rows_a, sem_a)

            pltpu.make_async_copy(xs.at[isrc_v.at[k2 + 1]], rows_b,
                                  sem_b).wait()
            pltpu.sync_copy(rows_b, agg_sh.at[idst_v.at[k2 + 1]], add=True)
            return carry2

        lax.fori_loop(0, _HNCH // 2, body, 0)
        return carry

    lax.fori_loop(0, _NCH // _HNCH, half, 0)
    plsc.subcore_barrier()
    _row_slab_copy(s, agg_sh, out.at[c])


_agg_call = pl.kernel(
    _agg_body,
    out_type=jax.ShapeDtypeStruct((_NC, _N, _D), jnp.float32),
    mesh=_sc_mesh,
    scratch_types=[
        pltpu.VMEM((_HNCH, _CH), jnp.int32),
        pltpu.VMEM((_HNCH, _CH), jnp.int32),
        pltpu.VMEM((_CH, _D), jnp.float32),
        pltpu.VMEM((_CH, _D), jnp.float32),
        pltpu.SemaphoreType.DMA,
        pltpu.SemaphoreType.DMA,
        pltpu.VMEM_SHARED((_N, _D), jnp.float32),
    ],
)


def _xs_body(x_ref, d0_ref, d1_ref, xs_ref):
    deg = jnp.maximum(d0_ref[...][:, :1] + d1_ref[...][:, :1], 1.0)
    xs_ref[...] = x_ref[...] * lax.rsqrt(deg)


def _xs_scale(x, d0, d1):
    return pl.pallas_call(
        _xs_body,
        grid=(_NB,),
        in_specs=[
            pl.BlockSpec((_B, _D), lambda i: (i, 0)),
            pl.BlockSpec((_B, _D), lambda i: (i, 0)),
            pl.BlockSpec((_B, _D), lambda i: (i, 0)),
        ],
        out_specs=pl.BlockSpec((_B, _D), lambda i: (i, 0)),
        out_shape=jax.ShapeDtypeStruct((_N, _D), jnp.float32),
    )(x, d0, d1)


def _mlp_block(x, W_ref, b_ref, a_ref):
    h = x
    for j in range(3):
        h = jnp.dot(h, W_ref[j], preferred_element_type=jnp.float32) \
            + b_ref[j:j + 1, :]
        a = a_ref[j:j + 1, :]
        h = jnp.where(h >= 0, h, a * h)
    return h + jnp.dot(x, W_ref[3], preferred_element_type=jnp.float32) \
        + b_ref[3:4, :]


def _enc_body(s0, s1, d0, d1, bm, encW, encb, LW, Lb, La, zl_ref, g_ref):
    i = pl.program_id(0)

    @pl.when(i == 0)
    def _init():
        g_ref[...] = jnp.zeros((_G, _D), jnp.float32)

    deg = jnp.maximum(d0[...][:, :1] + d1[...][:, :1], 1.0)
    rdeg = lax.rsqrt(deg)
    agg = (s0[...] + s1[...]) * rdeg
    z = jnp.maximum(
        jnp.dot(agg, encW[...], preferred_element_type=jnp.float32)
        + encb[...], 0.0)
    colid = lax.broadcasted_iota(jnp.int32, (_B, _G), 1)
    oh = (bm[...] == colid).astype(jnp.float32)
    g_ref[...] += lax.dot_general(oh, z, (((0,), (0,)), ((), ())),
                                  preferred_element_type=jnp.float32)
    zl_ref[...] = _mlp_block(z, LW, Lb, La)


def _enc_call(s0, s1, d0, d1, bm, encW, encb, LW, Lb, La):
    return pl.pallas_call(
        _enc_body,
        grid=(_NB,),
        in_specs=[
            pl.BlockSpec((_B, _D), lambda i: (i, 0)),
            pl.BlockSpec((_B, _D), lambda i: (i, 0)),
            pl.BlockSpec((_B, _D), lambda i: (i, 0)),
            pl.BlockSpec((_B, _D), lambda i: (i, 0)),
            pl.BlockSpec((_B, _G), lambda i: (i, 0)),
            pl.BlockSpec((_D, _D), lambda i: (0, 0)),
            pl.BlockSpec((1, _D), lambda i: (0, 0)),
            pl.BlockSpec((4, _D, _D), lambda i: (0, 0, 0)),
            pl.BlockSpec((4, _D), lambda i: (0, 0)),
            pl.BlockSpec((3, _D), lambda i: (0, 0)),
        ],
        out_specs=[
            pl.BlockSpec((_B, _D), lambda i: (i, 0)),
            pl.BlockSpec((_G, _D), lambda i: (0, 0)),
        ],
        out_shape=[
            jax.ShapeDtypeStruct((_N, _D), jnp.float32),
            jax.ShapeDtypeStruct((_G, _D), jnp.float32),
        ],
    )(s0, s1, d0, d1, bm, encW, encb, LW, Lb, La)


def _loss_body(zl, g, GW, Gb, Ga, bm, s1_ref, s2_ref, p1_ref, p2_ref, gg_s):
    i = pl.program_id(0)

    @pl.when(i == 0)
    def _init():
        gg_s[...] = _mlp_block(g[...], GW, Gb, Ga)
        s1_ref[...] = jnp.zeros((1, 1), jnp.float32)
        s2_ref[...] = jnp.zeros((1, 1), jnp.float32)
        p1_ref[...] = jnp.zeros((1, 1), jnp.float32)
        p2_ref[...] = jnp.zeros((1, 1), jnp.float32)

    r = lax.dot_general(zl[...], gg_s[...], (((1,), (1,)), ((), ())),
                        preferred_element_type=jnp.float32)
    colid = lax.broadcasted_iota(jnp.int32, (_B, _G), 1)
    pos = bm[...] == colid
    sp = jnp.maximum(-r, 0.0) + jnp.log1p(jnp.exp(-jnp.abs(r)))
    zero = jnp.zeros_like(r)
    s1_ref[...] += jnp.sum(sp, keepdims=True)
    s2_ref[...] += jnp.sum(r, keepdims=True)
    p1_ref[...] += jnp.sum(jnp.where(pos, sp, zero), keepdims=True)
    p2_ref[...] += jnp.sum(jnp.where(pos, r, zero), keepdims=True)


def _loss_call(zl, g, GW, Gb, Ga, bm):
    return pl.pallas_call(
        _loss_body,
        grid=(_NB,),
        in_specs=[
            pl.BlockSpec((_B, _D), lambda i: (i, 0)),
            pl.BlockSpec((_G, _D), lambda i: (0, 0)),
            pl.BlockSpec((4, _D, _D), lambda i: (0, 0, 0)),
            pl.BlockSpec((4, _D), lambda i: (0, 0)),
            pl.BlockSpec((3, _D), lambda i: (0, 0)),
            pl.BlockSpec((_B, _G), lambda i: (i, 0)),
        ],
        out_specs=[pl.BlockSpec((1, 1), lambda i: (0, 0))] * 4,
        out_shape=[jax.ShapeDtypeStruct((1, 1), jnp.float32)] * 4,
        scratch_shapes=[pltpu.VMEM((_G, _D), jnp.float32)],
    )(zl, g, GW, Gb, Ga, bm)


def _stack_mlp(p):
    W = jnp.stack([p["fc1"]["W"], p["fc2"]["W"], p["fc3"]["W"], p["sc"]["W"]])
    b = jnp.stack([p["fc1"]["b"], p["fc2"]["b"], p["fc3"]["b"], p["sc"]["b"]])
    a = jnp.stack([jnp.broadcast_to(p["a1"], (_D,)),
                   jnp.broadcast_to(p["a2"], (_D,)),
                   jnp.broadcast_to(p["a3"], (_D,))])
    return W, b, a


def _branch_loss(enc_p, Lwba, Gwba, x, edge_index, batch, ones128, zeros128,
                 prev):
    src2 = edge_index[0].reshape(_E // _CH, _CH)
    dst2 = edge_index[1].reshape(_E // _CH, _CH)
    if prev is not None:
        dst2, _ = lax.optimization_barrier((dst2, prev))
    deg2 = _deg_call(dst2, ones128, zeros128)
    d0, d1 = deg2[0], deg2[1]
    xs = _xs_scale(x, d0, d1)
    s2o = _agg_call(src2, dst2, xs, zeros128)
    bm = jnp.broadcast_to(batch[:, None], (_N, _G))
    encb = enc_p["b"].reshape(1, _D)
    zl, g = _enc_call(s2o[0], s2o[1], d0, d1, bm, enc_p["W"], encb, *Lwba)
    s1, s2, p1, p2 = [o[0, 0] for o in _loss_call(zl, g, *Gwba, bm)]

    e_pos = (_N * _LOG2 - p1) / _N
    e_neg = ((s1 + s2 - _N * _G * _LOG2) - (p1 + p2 - _N * _LOG2)) \
        / (_N * (_G - 1))
    return 2.0 * (e_neg - e_pos), s2o[0, :1, :1]


def kernel(x1, x2, x3, edge_index1, edge_index2, edge_index3,
           batch1, batch2, batch3, params):
    ones128 = jnp.ones((_CH, _D), jnp.float32)
    zeros128 = jnp.zeros((_N, _D), jnp.float32)
    Lwba = _stack_mlp(params["local"])
    Gwba = _stack_mlp(params["global"])
    l1, t1 = _branch_loss(params["enc1"], Lwba, Gwba, x1, edge_index1, batch1,
                          ones128, zeros128, None)
    l2, t2 = _branch_loss(params["enc2"], Lwba, Gwba, x2, edge_index2, batch2,
                          ones128, zeros128, t1)
    l3, _ = _branch_loss(params["enc3"], Lwba, Gwba, x3, edge_index3, batch3,
                         ones128, zeros128, t2)
    return l1 + l2 + l3
